# Initial kernel scaffold; baseline (speedup 1.0000x reference)
import jax
import jax.numpy as jnp
from jax.experimental import pallas as pl


def kernel(x, edge_index, edge_attr, W_src, W_dst, W_edge, att_src, att_dst, att_edge, bias):
    raise NotImplementedError("write your pallas kernel here")


# speedup vs baseline: 117.7971x; 117.7971x over previous
import functools

import jax
import jax.numpy as jnp
from jax import lax
from jax.experimental import pallas as pl
from jax.experimental.pallas import tpu as pltpu
from jax.experimental.pallas import tpu_sc as plsc

N = 10000
E = 320000
NC = 2
NS = 16
NW = NC * NS
CH = E // NW
L = 16
NEG = 0.2
TBL_CH = 1000
TBL_CHUNKS = 4 * N // TBL_CH
D_CH = 400
D_CHUNKS = N // D_CH


def _k1_body(x_ref, ws_ref, wd_ref, atts_ref, attd_ref, we_ref, atte_ref,
             tbl_ref, c_ref):
    xsT = lax.dot_general(ws_ref[...], x_ref[...],
                          (((1,), (1,)), ((), ())),
                          preferred_element_type=jnp.float32)
    vd = lax.dot_general(attd_ref[...], wd_ref[...],
                         (((1,), (0,)), ((), ())),
                         preferred_element_type=jnp.float32)
    adst = lax.dot_general(vd, x_ref[...],
                           (((1,), (1,)), ((), ())),
                           preferred_element_type=jnp.float32)
    tbl_ref[pl.ds(0, N)] = jnp.sum(xsT * atts_ref[...], axis=0)
    tbl_ref[pl.ds(N, N)] = adst[0]
    tbl_ref[pl.ds(2 * N, N)] = xsT[0]
    tbl_ref[pl.ds(3 * N, N)] = xsT[1]
    c = jnp.sum(we_ref[...] * atte_ref[...])
    c_ref[...] = jnp.broadcast_to(c, (L,))


_k1 = pl.pallas_call(
    _k1_body,
    out_shape=[
        jax.ShapeDtypeStruct((4 * N,), jnp.float32),
        jax.ShapeDtypeStruct((L,), jnp.float32),
    ],
)


_sc_mesh = plsc.VectorSubcoreMesh(core_axis_name="c", subcore_axis_name="s",
                                  num_cores=NC, num_subcores=NS)


@functools.partial(
    pl.kernel,
    mesh=_sc_mesh,
    compiler_params=pltpu.CompilerParams(needs_layout_passes=False),
    out_type=[
        jax.ShapeDtypeStruct((E,), jnp.float32),
        jax.ShapeDtypeStruct((NW, L), jnp.float32),
        jax.ShapeDtypeStruct((NW, N), jnp.float32),
        jax.ShapeDtypeStruct((NW, N), jnp.float32),
        jax.ShapeDtypeStruct((NW, N), jnp.float32),
    ],
    scratch_types=[
        pltpu.VMEM((4 * N,), jnp.float32),
        pltpu.VMEM((N,), jnp.float32),
        pltpu.VMEM((N,), jnp.float32),
        pltpu.VMEM((N,), jnp.float32),
        pltpu.VMEM((CH,), jnp.int32),
        pltpu.VMEM((CH,), jnp.int32),
        pltpu.VMEM((CH,), jnp.float32),
        pltpu.VMEM((L,), jnp.float32),
        pltpu.VMEM((L,), jnp.float32),
        pltpu.SemaphoreType.DMA,
    ],
)
def _kab(tbl_h, ei_h, ea_h, c_h,
         p_h, gw_h, dw_h, o0_h, o1_h,
         t_all, acc_d, acc_0, acc_1,
         e_src, e_dst, e_val, cbuf, gbuf, sem):
    wid = lax.axis_index("c") * NS + lax.axis_index("s")
    base = wid * CH
    handles = [
        pltpu.async_copy(ei_h.at[pl.ds(base, CH)], e_src, sem),
        pltpu.async_copy(ei_h.at[pl.ds(E + base, CH)], e_dst, sem),
        pltpu.async_copy(ea_h.at[pl.ds(base, CH)], e_val, sem),
        pltpu.async_copy(c_h, cbuf, sem),
    ]
    start = (wid * TBL_CHUNKS) // NW
    for k in range(TBL_CHUNKS):
        j = start + k
        j = jnp.where(j >= TBL_CHUNKS, j - TBL_CHUNKS, j)
        off = j * TBL_CH
        handles.append(pltpu.async_copy(
            tbl_h.at[pl.ds(off, TBL_CH)], t_all.at[pl.ds(off, TBL_CH)], sem))

    zero = jnp.zeros((L,), jnp.float32)

    def zbody(i, carry):
        dsl = pl.ds(i * L, L)
        acc_d[dsl] = zero
        acc_0[dsl] = zero
        acc_1[dsl] = zero
        return carry

    lax.fori_loop(0, N // L, zbody, 0)
    for h in handles:
        h.wait()

    c = cbuf[...][0]

    def body_a(i, gmax):
        dsl = pl.ds(i * L, L)
        s = e_src[dsl]
        d = e_dst[dsl]
        ev = e_val[dsl]
        av = plsc.load_gather(t_all, [s])
        bv = plsc.load_gather(t_all, [d + N])
        lgt = av + bv + c * ev
        lgt = jnp.where(lgt >= 0.0, lgt, lgt * NEG)
        e_val[dsl] = lgt
        return jnp.maximum(gmax, lgt)

    gmax = lax.fori_loop(0, CH // L, body_a,
                         jnp.full((L,), -3.4e38, jnp.float32))
    g_w = jnp.max(gmax)
    gbuf[...] = jnp.broadcast_to(g_w, (L,))
    pltpu.sync_copy(gbuf, gw_h.at[wid])

    def body_b(i, carry):
        dsl = pl.ds(i * L, L)
        lgt = e_val[dsl]
        p = jnp.exp(lgt - g_w)
        e_val[dsl] = p
        s = e_src[dsl]
        d = e_dst[dsl]
        plsc.addupdate_scatter(acc_d, [d], p)
        x0 = plsc.load_gather(t_all, [s + 2 * N])
        x1 = plsc.load_gather(t_all, [s + 3 * N])
        plsc.addupdate_scatter(acc_0, [d], p * x0)
        plsc.addupdate_scatter(acc_1, [d], p * x1)
        return carry

    lax.fori_loop(0, CH // L, body_b, 0)

    pltpu.sync_copy(e_val, p_h.at[pl.ds(base, CH)])
    pltpu.sync_copy(acc_d, dw_h.at[wid])
    pltpu.sync_copy(acc_0, o0_h.at[wid])
    pltpu.sync_copy(acc_1, o1_h.at[wid])


def _k2_body(gw_ref, dw_ref, o0_ref, o1_ref, b_ref,
             out_ref, d_ref, s_ref):
    gwc = gw_ref[:, 0:1]
    g = jnp.max(gwc)
    s = jnp.exp(gwc - g)
    D = jnp.sum(dw_ref[...] * s, axis=0, keepdims=True)
    O0 = jnp.sum(o0_ref[...] * s, axis=0, keepdims=True)
    O1 = jnp.sum(o1_ref[...] * s, axis=0, keepdims=True)
    valid = D > 0.0
    Dsafe = jnp.where(valid, D, 1.0)
    out0 = jnp.where(valid, O0 / Dsafe, 0.0) + b_ref[0:1, :]
    out1 = jnp.where(valid, O1 / Dsafe, 0.0) + b_ref[1:2, :]
    out_ref[...] = jnp.concatenate([out0, out1], axis=0).T
    d_ref[...] = D[0]
    s_ref[...] = jnp.broadcast_to(s, (NW, L))


_k2 = pl.pallas_call(
    _k2_body,
    out_shape=[
        jax.ShapeDtypeStruct((N, 2), jnp.float32),
        jax.ShapeDtypeStruct((N,), jnp.float32),
        jax.ShapeDtypeStruct((NW, L), jnp.float32),
    ],
)


@functools.partial(
    pl.kernel,
    mesh=_sc_mesh,
    compiler_params=pltpu.CompilerParams(needs_layout_passes=False),
    out_type=[jax.ShapeDtypeStruct((E,), jnp.float32)],
    scratch_types=[
        pltpu.VMEM((N,), jnp.float32),
        pltpu.VMEM((CH,), jnp.float32),
        pltpu.VMEM((CH,), jnp.int32),
        pltpu.VMEM((L,), jnp.float32),
        pltpu.SemaphoreType.DMA,
    ],
)
def _kc(p_h, ei_h, d_h, s_h, alpha_h, t_d, e_p, e_d, sbuf, sem):
    wid = lax.axis_index("c") * NS + lax.axis_index("s")
    base = wid * CH
    handles = [
        pltpu.async_copy(p_h.at[pl.ds(base, CH)], e_p, sem),
        pltpu.async_copy(ei_h.at[pl.ds(E + base, CH)], e_d, sem),
        pltpu.async_copy(s_h.at[wid], sbuf, sem),
    ]
    start = (wid * D_CHUNKS) // NW
    for k in range(D_CHUNKS):
        j = start + k
        j = jnp.where(j >= D_CHUNKS, j - D_CHUNKS, j)
        off = j * D_CH
        handles.append(pltpu.async_copy(
            d_h.at[pl.ds(off, D_CH)], t_d.at[pl.ds(off, D_CH)], sem))
    for h in handles:
        h.wait()
    sw = sbuf[...][0]

    def body(i, carry):
        dsl = pl.ds(i * L, L)
        pv = e_p[dsl]
        dv = e_d[dsl]
        Dv = plsc.load_gather(t_d, [dv])
        e_p[dsl] = pv * sw / Dv
        return carry

    lax.fori_loop(0, CH // L, body, 0)
    pltpu.sync_copy(e_p, alpha_h.at[pl.ds(base, CH)])


def kernel(x, edge_index, edge_attr, W_src, W_dst, W_edge,
           att_src, att_dst, att_edge, bias):
    ei_flat = edge_index.reshape(2 * E)
    atts = att_src.reshape(2, 1)
    attd = att_dst.reshape(1, 2)
    atte = att_edge.reshape(1, 2)
    we = W_edge.reshape(1, 2)

    tbl, cvec = _k1(x, W_src, W_dst, atts, attd, we, atte)
    p, gw, dw, o0w, o1w = _kab(tbl, ei_flat, edge_attr.reshape(E), cvec)
    out, D, sout = _k2(gw, dw, o0w, o1w, bias.reshape(2, 1))
    (alpha,) = _kc(p, ei_flat, D, sout)
    return out, alpha.reshape(E, 1)


# speedup vs baseline: 154.5074x; 1.3116x over previous
import functools

import jax
import jax.numpy as jnp
from jax import lax
from jax.experimental import pallas as pl
from jax.experimental.pallas import tpu as pltpu
from jax.experimental.pallas import tpu_sc as plsc

N = 10000
E = 320000
NC = 2
NS = 16
NW = NC * NS
L = 16
CH = 9984
NTAIL = E - NW * CH
TAIL = NW * CH
CHB = CH + NTAIL
NEG = 0.2
TBL_CH = 1000
TBL_CHUNKS = 4 * N // TBL_CH
D_CH = 400
D_CHUNKS = N // D_CH


def _k1_body(x_ref, ws_ref, wd_ref, atts_ref, attd_ref, we_ref, atte_ref,
             tbl_ref, c_ref):
    xsT = lax.dot_general(ws_ref[...], x_ref[...],
                          (((1,), (1,)), ((), ())),
                          preferred_element_type=jnp.float32)
    vd = lax.dot_general(attd_ref[...], wd_ref[...],
                         (((1,), (0,)), ((), ())),
                         preferred_element_type=jnp.float32)
    adst = lax.dot_general(vd, x_ref[...],
                           (((1,), (1,)), ((), ())),
                           preferred_element_type=jnp.float32)
    tbl_ref[pl.ds(0, N)] = jnp.sum(xsT * atts_ref[...].T, axis=0)
    tbl_ref[pl.ds(N, N)] = adst[0]
    tbl_ref[pl.ds(2 * N, N)] = xsT[0]
    tbl_ref[pl.ds(3 * N, N)] = xsT[1]
    c = jnp.sum(we_ref[...] * atte_ref[...])
    c_ref[...] = jnp.broadcast_to(c, (L,))


_k1 = pl.pallas_call(
    _k1_body,
    out_shape=[
        jax.ShapeDtypeStruct((4 * N,), jnp.float32),
        jax.ShapeDtypeStruct((L,), jnp.float32),
    ],
)


_sc_mesh = plsc.VectorSubcoreMesh(core_axis_name="c", subcore_axis_name="s",
                                  num_cores=NC, num_subcores=NS)


@functools.partial(
    pl.kernel,
    mesh=_sc_mesh,
    compiler_params=pltpu.CompilerParams(needs_layout_passes=False),
    out_type=[
        jax.ShapeDtypeStruct((E,), jnp.float32),
        jax.ShapeDtypeStruct((NW, L), jnp.float32),
        jax.ShapeDtypeStruct((NW, N), jnp.float32),
        jax.ShapeDtypeStruct((NW, N), jnp.float32),
        jax.ShapeDtypeStruct((NW, N), jnp.float32),
    ],
    scratch_types=[
        pltpu.VMEM((4 * N,), jnp.float32),
        pltpu.VMEM((N,), jnp.float32),
        pltpu.VMEM((N,), jnp.float32),
        pltpu.VMEM((N,), jnp.float32),
        pltpu.VMEM((CHB,), jnp.int32),
        pltpu.VMEM((CHB,), jnp.int32),
        pltpu.VMEM((CHB,), jnp.float32),
        pltpu.VMEM((L,), jnp.float32),
        pltpu.VMEM((L,), jnp.float32),
        pltpu.SemaphoreType.DMA,
    ],
)
def _kab(tbl_h, ei_h, ea_h, c_h,
         p_h, gw_h, dw_h, o0_h, o1_h,
         t_all, acc_d, acc_0, acc_1,
         e_src, e_dst, e_val, cbuf, gbuf, sem):
    wid = lax.axis_index("c") * NS + lax.axis_index("s")
    base = wid * CH
    handles = [
        pltpu.async_copy(ei_h.at[pl.ds(base, CH)], e_src.at[pl.ds(0, CH)], sem),
        pltpu.async_copy(ei_h.at[pl.ds(E + base, CH)], e_dst.at[pl.ds(0, CH)], sem),
        pltpu.async_copy(ea_h.at[0, pl.ds(base, CH)], e_val.at[pl.ds(0, CH)], sem),
        pltpu.async_copy(c_h, cbuf, sem),
        pltpu.async_copy(ei_h.at[pl.ds(TAIL, NTAIL)],
                         e_src.at[pl.ds(CH, NTAIL)], sem),
        pltpu.async_copy(ei_h.at[pl.ds(E + TAIL, NTAIL)],
                         e_dst.at[pl.ds(CH, NTAIL)], sem),
        pltpu.async_copy(ea_h.at[0, pl.ds(TAIL, NTAIL)],
                         e_val.at[pl.ds(CH, NTAIL)], sem),
    ]
    start = (wid * TBL_CHUNKS) // NW
    for k in range(TBL_CHUNKS):
        j = start + k
        j = jnp.where(j >= TBL_CHUNKS, j - TBL_CHUNKS, j)
        off = j * TBL_CH
        handles.append(pltpu.async_copy(
            tbl_h.at[pl.ds(off, TBL_CH)], t_all.at[pl.ds(off, TBL_CH)], sem))

    zero = jnp.zeros((L,), jnp.float32)

    def zbody(i, carry):
        dsl = pl.ds(i * L, L)
        acc_d[dsl] = zero
        acc_0[dsl] = zero
        acc_1[dsl] = zero
        return carry

    lax.fori_loop(0, N // L, zbody, 0)
    for h in handles:
        h.wait()

    c = cbuf[...][0]

    def body_a(i, gmax):
        dsl = pl.ds(i * L, L)
        s = e_src[dsl]
        d = e_dst[dsl]
        ev = e_val[dsl]
        av = plsc.load_gather(t_all, [s])
        bv = plsc.load_gather(t_all, [d + N])
        lgt = av + bv + c * ev
        lgt = jnp.where(lgt >= 0.0, lgt, lgt * NEG)
        e_val[dsl] = lgt
        return jnp.maximum(gmax, lgt)

    gmax = lax.fori_loop(0, CH // L, body_a,
                         jnp.full((L,), -3.4e38, jnp.float32))
    gbuf[...] = gmax

    @pl.when(wid == 0)
    def _tail_a():
        def tbody(i, carry):
            dsl = pl.ds(i * L, L)
            s = e_src[dsl]
            d = e_dst[dsl]
            ev = e_val[dsl]
            av = plsc.load_gather(t_all, [s])
            bv = plsc.load_gather(t_all, [d + N])
            lgt = av + bv + c * ev
            lgt = jnp.where(lgt >= 0.0, lgt, lgt * NEG)
            e_val[dsl] = lgt
            gbuf[...] = jnp.maximum(gbuf[...], lgt)
            return carry

        lax.fori_loop(CH // L, CHB // L, tbody, 0)

    g_w = jnp.max(gbuf[...])

    def body_b(i, carry):
        dsl = pl.ds(i * L, L)
        lgt = e_val[dsl]
        p = jnp.exp(lgt - g_w)
        e_val[dsl] = p
        s = e_src[dsl]
        d = e_dst[dsl]
        plsc.addupdate_scatter(acc_d, [d], p)
        x0 = plsc.load_gather(t_all, [s + 2 * N])
        x1 = plsc.load_gather(t_all, [s + 3 * N])
        plsc.addupdate_scatter(acc_0, [d], p * x0)
        plsc.addupdate_scatter(acc_1, [d], p * x1)
        return carry

    lax.fori_loop(0, CH // L, body_b, 0)

    @pl.when(wid == 0)
    def _tail_b():
        lax.fori_loop(CH // L, CHB // L, body_b, 0)

    gbuf[...] = jnp.broadcast_to(g_w, (L,))
    pltpu.sync_copy(gbuf, gw_h.at[wid])

    pltpu.sync_copy(e_val.at[pl.ds(0, CH)], p_h.at[pl.ds(base, CH)])

    @pl.when(wid == 0)
    def _tail_p():
        pltpu.sync_copy(e_val.at[pl.ds(CH, NTAIL)], p_h.at[pl.ds(TAIL, NTAIL)])

    pltpu.sync_copy(acc_d, dw_h.at[wid])
    pltpu.sync_copy(acc_0, o0_h.at[wid])
    pltpu.sync_copy(acc_1, o1_h.at[wid])


def _k2_body(gw_ref, dw_ref, o0_ref, o1_ref, b_ref,
             out_ref, d_ref, s_ref):
    gwc = gw_ref[:, 0:1]
    g = jnp.max(gwc)
    s = jnp.exp(gwc - g)
    D = jnp.sum(dw_ref[...] * s, axis=0, keepdims=True)
    O0 = jnp.sum(o0_ref[...] * s, axis=0, keepdims=True)
    O1 = jnp.sum(o1_ref[...] * s, axis=0, keepdims=True)
    valid = D > 0.0
    Dinv = jnp.where(valid, 1.0 / jnp.where(valid, D, 1.0), 0.0)
    out0 = O0 * Dinv + b_ref[0:1, :]
    out1 = O1 * Dinv + b_ref[1:2, :]
    out_ref[...] = jnp.concatenate([out0, out1], axis=0).T
    d_ref[...] = Dinv[0]
    s_ref[...] = jnp.broadcast_to(s, (NW, L))


_k2 = pl.pallas_call(
    _k2_body,
    out_shape=[
        jax.ShapeDtypeStruct((N, 2), jnp.float32),
        jax.ShapeDtypeStruct((N,), jnp.float32),
        jax.ShapeDtypeStruct((NW, L), jnp.float32),
    ],
)


@functools.partial(
    pl.kernel,
    mesh=_sc_mesh,
    compiler_params=pltpu.CompilerParams(needs_layout_passes=False),
    out_type=[jax.ShapeDtypeStruct((1, E), jnp.float32)],
    scratch_types=[
        pltpu.VMEM((N,), jnp.float32),
        pltpu.VMEM((CHB,), jnp.float32),
        pltpu.VMEM((CHB,), jnp.int32),
        pltpu.VMEM((L,), jnp.float32),
        pltpu.SemaphoreType.DMA,
    ],
)
def _kc(p_h, ei_h, d_h, s_h, alpha_h, t_d, e_p, e_d, sbuf, sem):
    wid = lax.axis_index("c") * NS + lax.axis_index("s")
    base = wid * CH
    handles = [
        pltpu.async_copy(p_h.at[pl.ds(base, CH)], e_p.at[pl.ds(0, CH)], sem),
        pltpu.async_copy(ei_h.at[pl.ds(E + base, CH)], e_d.at[pl.ds(0, CH)], sem),
        pltpu.async_copy(s_h.at[wid], sbuf, sem),
        pltpu.async_copy(p_h.at[pl.ds(TAIL, NTAIL)],
                         e_p.at[pl.ds(CH, NTAIL)], sem),
        pltpu.async_copy(ei_h.at[pl.ds(E + TAIL, NTAIL)],
                         e_d.at[pl.ds(CH, NTAIL)], sem),
    ]
    start = (wid * D_CHUNKS) // NW
    for k in range(D_CHUNKS):
        j = start + k
        j = jnp.where(j >= D_CHUNKS, j - D_CHUNKS, j)
        off = j * D_CH
        handles.append(pltpu.async_copy(
            d_h.at[pl.ds(off, D_CH)], t_d.at[pl.ds(off, D_CH)], sem))
    for h in handles:
        h.wait()
    sw = sbuf[...][0]

    def body(i, carry):
        dsl = pl.ds(i * L, L)
        pv = e_p[dsl]
        dv = e_d[dsl]
        Dv = plsc.load_gather(t_d, [dv])
        e_p[dsl] = pv * sw * Dv
        return carry

    lax.fori_loop(0, CH // L, body, 0)

    @pl.when(wid == 0)
    def _tail():
        lax.fori_loop(CH // L, CHB // L, body, 0)

    pltpu.sync_copy(e_p.at[pl.ds(0, CH)], alpha_h.at[0, pl.ds(base, CH)])

    @pl.when(wid == 0)
    def _tail_w():
        pltpu.sync_copy(e_p.at[pl.ds(CH, NTAIL)],
                        alpha_h.at[0, pl.ds(TAIL, NTAIL)])


def kernel(x, edge_index, edge_attr, W_src, W_dst, W_edge,
           att_src, att_dst, att_edge, bias):
    ei_flat = edge_index.reshape(2 * E)
    ea_row = edge_attr.T
    atts = att_src.reshape(1, 2)
    attd = att_dst.reshape(1, 2)
    atte = att_edge.reshape(1, 2)

    tbl, cvec = _k1(x, W_src, W_dst, atts, attd, W_edge.reshape(1, 2), atte)
    p, gw, dw, o0w, o1w = _kab(tbl, ei_flat, ea_row, cvec)
    out, Dinv, sout = _k2(gw, dw, o0w, o1w, bias.reshape(2, 1))
    (alpha,) = _kc(p, ei_flat, Dinv, sout)
    return out, alpha.T
